# baseline (speedup 1.0000x reference)
import jax
import jax.numpy as jnp
from jax import lax
from jax.experimental import pallas as pl
from jax.experimental.pallas import tpu as pltpu
from jax.experimental.pallas import tpu_sc as plsc

_N = 10000
_D = 128
_H = 64
_E = 320000
_ITERS = 4

_NC = 2
_NS = 16
_NW = _NC * _NS
_CH = 128
_NB = 8
_DEPTH = 4
_EPT = -(-_E // _NW)
_GROUPS = -(-_EPT // (_CH * _NB))
_CHUNKS = _GROUPS * _NB
_E_PAD = _CHUNKS * _CH * _NW
_NPAD = 10112
_RPT = _NPAD // _NS


def _sc_agg_body(m_hbm, src_hbm, dst_hbm, out0_hbm, out1_hbm,
                 idx_s, idx_d, rows, acc, gsems, ssems):
    c = lax.axis_index("c")
    s = lax.axis_index("s")
    wid = s * _NC + c
    row0 = s * _RPT
    rem = _RPT - 4 * _CH

    pltpu.sync_copy(src_hbm.at[wid], idx_s)
    pltpu.sync_copy(dst_hbm.at[wid], idx_d)

    zv = jnp.zeros((16,), jnp.float32)

    def _zero_row(i, carry):
        for j in range(_H // 16):
            rows[0, i, pl.ds(j * 16, 16)] = zv
        return carry

    lax.fori_loop(0, _CH, _zero_row, 0)
    for t in range(4):
        pltpu.sync_copy(rows.at[0], acc.at[pl.ds(row0 + t * _CH, _CH)])
    pltpu.sync_copy(rows.at[0].at[pl.ds(0, rem)],
                    acc.at[pl.ds(row0 + 4 * _CH, rem)])
    plsc.subcore_barrier()

    def _gather(j, b):
        pltpu.async_copy(m_hbm.at[idx_s.at[j]], rows.at[b], gsems.at[b])

    def _gwait(j, b):
        pltpu.make_async_copy(
            m_hbm.at[idx_s.at[j]], rows.at[b], gsems.at[b]).wait()

    def _swait(j, b):
        pltpu.make_async_copy(
            rows.at[b], acc.at[idx_d.at[j]], ssems.at[b]).wait()

    for k in range(_DEPTH):
        _gather(k, k)

    def _group(g, carry):
        for k in range(_NB):
            j = g * _NB + k
            bn = (k + _DEPTH) % _NB

            @pl.when(j >= _DEPTH)
            def _():
                _swait(j - _DEPTH, bn)

            @pl.when(j + _DEPTH < _CHUNKS)
            def _():
                _gather(j + _DEPTH, bn)

            _gwait(j, k)
            pltpu.async_copy(rows.at[k], acc.at[idx_d.at[j]], ssems.at[k],
                             add=True)
        return carry

    lax.fori_loop(0, _GROUPS, _group, 0)
    for k in range(_DEPTH):
        _swait(_CHUNKS - _DEPTH + k, (_CHUNKS - _DEPTH + k) % _NB)
    plsc.subcore_barrier()

    for t in range(4):
        pltpu.sync_copy(acc.at[pl.ds(row0 + t * _CH, _CH)], rows.at[t])
    pltpu.sync_copy(acc.at[pl.ds(row0 + 4 * _CH, rem)],
                    rows.at[4].at[pl.ds(0, rem)])

    @pl.when(c == 0)
    def _():
        for t in range(4):
            pltpu.sync_copy(rows.at[t], out0_hbm.at[pl.ds(row0 + t * _CH, _CH)])
        pltpu.sync_copy(rows.at[4].at[pl.ds(0, rem)],
                        out0_hbm.at[pl.ds(row0 + 4 * _CH, rem)])

    @pl.when(c == 1)
    def _():
        for t in range(4):
            pltpu.sync_copy(rows.at[t], out1_hbm.at[pl.ds(row0 + t * _CH, _CH)])
        pltpu.sync_copy(rows.at[4].at[pl.ds(0, rem)],
                        out1_hbm.at[pl.ds(row0 + 4 * _CH, rem)])


_sc_agg_cache = []


def _sc_agg(m, srcp, dstp):
    if not _sc_agg_cache:
        _sc_agg_cache.append(pl.kernel(
            _sc_agg_body,
            out_type=(jax.ShapeDtypeStruct((_NPAD, _H), jnp.float32),
                      jax.ShapeDtypeStruct((_NPAD, _H), jnp.float32)),
            mesh=plsc.VectorSubcoreMesh(core_axis_name="c",
                                        subcore_axis_name="s"),
            scratch_types=[
                pltpu.VMEM((_CHUNKS, _CH), jnp.int32),
                pltpu.VMEM((_CHUNKS, _CH), jnp.int32),
                pltpu.VMEM((_NB, _CH, _H), jnp.float32),
                pltpu.VMEM_SHARED((_NPAD, _H), jnp.float32),
                pltpu.SemaphoreType.DMA((_NB,)),
                pltpu.SemaphoreType.DMA((_NB,)),
            ],
            compiler_params=pltpu.CompilerParams(use_tc_tiling_on_sc=False),
        ))
    return _sc_agg_cache[0](m, srcp, dstp)


def _enc_body(x_ref, wi_ref, bi_ref, wm_ref, bm_ref, h_ref, m_ref):
    h = jnp.maximum(
        jnp.dot(x_ref[...], wi_ref[...], preferred_element_type=jnp.float32)
        + bi_ref[...], 0.0)
    h_ref[...] = h
    m_ref[...] = jnp.maximum(
        jnp.dot(h, wm_ref[...], preferred_element_type=jnp.float32)
        + bm_ref[...], 0.0)


def _upd_body(h_ref, a0_ref, a1_ref, wt_ref, wb_ref, bu_ref, wm_ref, bm_ref,
              h_out, m_out):
    agg = a0_ref[...] + a1_ref[...]
    hn = jnp.maximum(
        jnp.dot(h_ref[...], wt_ref[...], preferred_element_type=jnp.float32)
        + jnp.dot(agg, wb_ref[...], preferred_element_type=jnp.float32)
        + bu_ref[...], 0.0)
    h_out[...] = hn
    m_out[...] = jnp.maximum(
        jnp.dot(hn, wm_ref[...], preferred_element_type=jnp.float32)
        + bm_ref[...], 0.0)


def _fin_body(h_ref, a0_ref, a1_ref, wt_ref, wb_ref, bu_ref, wo_ref, bo_ref,
              out_ref):
    agg = a0_ref[...] + a1_ref[...]
    hn = jnp.maximum(
        jnp.dot(h_ref[...], wt_ref[...], preferred_element_type=jnp.float32)
        + jnp.dot(agg, wb_ref[...], preferred_element_type=jnp.float32)
        + bu_ref[...], 0.0)
    out_ref[...] = (
        jnp.dot(hn, wo_ref[...], preferred_element_type=jnp.float32)
        + bo_ref[...])


def _hm_shapes():
    return (jax.ShapeDtypeStruct((_NPAD, _H), jnp.float32),
            jax.ShapeDtypeStruct((_NPAD, _H), jnp.float32))


def kernel(x, edge_index, W_in, b_in, W_msg, b_msg, W_upd, b_upd, W_out, b_out):
    f32 = jnp.float32
    xp = jnp.zeros((_NPAD, _D), f32).at[:_N, :].set(x)
    src = edge_index[0]
    dst = edge_index[1]
    srcp = jnp.concatenate(
        [src, jnp.zeros((_E_PAD - _E,), jnp.int32)]).reshape(
            _NW, _CHUNKS, _CH)
    dstp = jnp.concatenate(
        [dst, jnp.full((_E_PAD - _E,), _N, jnp.int32)]).reshape(_NW, _CHUNKS, _CH)
    bi = b_in.reshape(1, _H)
    bm = b_msg.reshape(1, _H)
    bu = b_upd.reshape(1, _H)
    bo = b_out.reshape(1, _H)
    wt = W_upd[:_H]
    wb = W_upd[_H:]

    h, m = pl.pallas_call(_enc_body, out_shape=_hm_shapes())(
        xp, W_in, bi, W_msg, bm)
    out = None
    for it in range(_ITERS):
        a0, a1 = _sc_agg(m, srcp, dstp)
        if it < _ITERS - 1:
            h, m = pl.pallas_call(_upd_body, out_shape=_hm_shapes())(
                h, a0, a1, wt, wb, bu, W_msg, bm)
        else:
            out = pl.pallas_call(
                _fin_body,
                out_shape=jax.ShapeDtypeStruct((_NPAD, _H), f32))(
                    h, a0, a1, wt, wb, bu, W_out, bo)
    return out[:_N]


# speedup vs baseline: 1.0927x; 1.0927x over previous
import jax
import jax.numpy as jnp
from jax import lax
from jax.experimental import pallas as pl
from jax.experimental.pallas import tpu as pltpu
from jax.experimental.pallas import tpu_sc as plsc

_N = 10000
_D = 128
_H = 64
_E = 320000
_ITERS = 4

_NC = 2
_NS = 16
_NW = _NC * _NS
_CH = 128
_NB = 4
_DEPTH = 2
_EPT = -(-_E // _NW)
_GROUPS = -(-_EPT // (_CH * _NB))
_CHUNKS = _GROUPS * _NB
_E_PAD = _CHUNKS * _CH * _NW
_NPAD = 10112
_RPT = _NPAD // _NS


def _sc_agg_body(m_hbm, src_hbm, dst_hbm, out0_hbm, out1_hbm,
                 idx_s, idx_d, rows, acc, gsems, ssems):
    c = lax.axis_index("c")
    s = lax.axis_index("s")
    wid = s * _NC + c
    row0 = s * _RPT
    rem = _RPT - 4 * _CH

    pltpu.sync_copy(src_hbm.at[wid], idx_s)
    pltpu.sync_copy(dst_hbm.at[wid], idx_d)

    zv = jnp.zeros((16,), jnp.float32)

    def _zero_row(i, carry):
        for j in range(_H // 16):
            rows[0, i, pl.ds(j * 16, 16)] = zv
        return carry

    lax.fori_loop(0, _CH, _zero_row, 0)
    for t in range(4):
        pltpu.sync_copy(rows.at[0], acc.at[pl.ds(row0 + t * _CH, _CH)])
    pltpu.sync_copy(rows.at[0].at[pl.ds(0, rem)],
                    acc.at[pl.ds(row0 + 4 * _CH, rem)])
    plsc.subcore_barrier()

    def _gather(j, b):
        pltpu.async_copy(m_hbm.at[idx_s.at[j]], rows.at[b], gsems.at[b])

    def _gwait(j, b):
        pltpu.make_async_copy(
            m_hbm.at[idx_s.at[j]], rows.at[b], gsems.at[b]).wait()

    def _swait(j, b):
        pltpu.make_async_copy(
            rows.at[b], acc.at[idx_d.at[j]], ssems.at[b]).wait()

    for k in range(_DEPTH):
        _gather(k, k)

    def _group(g, carry):
        for k in range(_NB):
            j = g * _NB + k
            bn = (k + _DEPTH) % _NB

            @pl.when(j >= _DEPTH)
            def _():
                _swait(j - _DEPTH, bn)

            @pl.when(j + _DEPTH < _CHUNKS)
            def _():
                _gather(j + _DEPTH, bn)

            _gwait(j, k)
            pltpu.async_copy(rows.at[k], acc.at[idx_d.at[j]], ssems.at[k],
                             add=True)
        return carry

    lax.fori_loop(0, _GROUPS, _group, 0)
    for k in range(_DEPTH):
        _swait(_CHUNKS - _DEPTH + k, (_CHUNKS - _DEPTH + k) % _NB)
    plsc.subcore_barrier()

    for t in range(4):
        pltpu.sync_copy(acc.at[pl.ds(row0 + t * _CH, _CH)], rows.at[t % _NB])

    @pl.when(c == 0)
    def _():
        for t in range(4):
            pltpu.sync_copy(rows.at[t % _NB],
                            out0_hbm.at[pl.ds(row0 + t * _CH, _CH)])

    @pl.when(c == 1)
    def _():
        for t in range(4):
            pltpu.sync_copy(rows.at[t % _NB],
                            out1_hbm.at[pl.ds(row0 + t * _CH, _CH)])

    pltpu.sync_copy(acc.at[pl.ds(row0 + 4 * _CH, rem)],
                    rows.at[0].at[pl.ds(0, rem)])

    @pl.when(c == 0)
    def _():
        pltpu.sync_copy(rows.at[0].at[pl.ds(0, rem)],
                        out0_hbm.at[pl.ds(row0 + 4 * _CH, rem)])

    @pl.when(c == 1)
    def _():
        pltpu.sync_copy(rows.at[0].at[pl.ds(0, rem)],
                        out1_hbm.at[pl.ds(row0 + 4 * _CH, rem)])


_sc_agg_cache = []


def _sc_agg(m, srcp, dstp):
    if not _sc_agg_cache:
        _sc_agg_cache.append(pl.kernel(
            _sc_agg_body,
            out_type=(jax.ShapeDtypeStruct((_NPAD, _H), jnp.float32),
                      jax.ShapeDtypeStruct((_NPAD, _H), jnp.float32)),
            mesh=plsc.VectorSubcoreMesh(core_axis_name="c",
                                        subcore_axis_name="s"),
            scratch_types=[
                pltpu.VMEM((_CHUNKS, _CH), jnp.int32),
                pltpu.VMEM((_CHUNKS, _CH), jnp.int32),
                pltpu.VMEM((_NB, _CH, _H), jnp.float32),
                pltpu.VMEM_SHARED((_NPAD, _H), jnp.float32),
                pltpu.SemaphoreType.DMA((_NB,)),
                pltpu.SemaphoreType.DMA((_NB,)),
            ],
            compiler_params=pltpu.CompilerParams(use_tc_tiling_on_sc=False),
        ))
    return _sc_agg_cache[0](m, srcp, dstp)


def _enc_body(x_ref, wi_ref, bi_ref, wm_ref, bm_ref, h_ref, m_ref):
    h = jnp.maximum(
        jnp.dot(x_ref[...], wi_ref[...], preferred_element_type=jnp.float32)
        + bi_ref[...], 0.0)
    h_ref[...] = h
    m_ref[...] = jnp.maximum(
        jnp.dot(h, wm_ref[...], preferred_element_type=jnp.float32)
        + bm_ref[...], 0.0)


def _upd_body(h_ref, a0_ref, a1_ref, wt_ref, wb_ref, bu_ref, wm_ref, bm_ref,
              h_out, m_out):
    agg = a0_ref[...] + a1_ref[...]
    hn = jnp.maximum(
        jnp.dot(h_ref[...], wt_ref[...], preferred_element_type=jnp.float32)
        + jnp.dot(agg, wb_ref[...], preferred_element_type=jnp.float32)
        + bu_ref[...], 0.0)
    h_out[...] = hn
    m_out[...] = jnp.maximum(
        jnp.dot(hn, wm_ref[...], preferred_element_type=jnp.float32)
        + bm_ref[...], 0.0)


def _fin_body(h_ref, a0_ref, a1_ref, wt_ref, wb_ref, bu_ref, wo_ref, bo_ref,
              out_ref):
    agg = a0_ref[...] + a1_ref[...]
    hn = jnp.maximum(
        jnp.dot(h_ref[...], wt_ref[...], preferred_element_type=jnp.float32)
        + jnp.dot(agg, wb_ref[...], preferred_element_type=jnp.float32)
        + bu_ref[...], 0.0)
    out_ref[...] = (
        jnp.dot(hn, wo_ref[...], preferred_element_type=jnp.float32)
        + bo_ref[...])


def _hm_shapes():
    return (jax.ShapeDtypeStruct((_NPAD, _H), jnp.float32),
            jax.ShapeDtypeStruct((_NPAD, _H), jnp.float32))


def kernel(x, edge_index, W_in, b_in, W_msg, b_msg, W_upd, b_upd, W_out, b_out):
    f32 = jnp.float32
    xp = jnp.zeros((_NPAD, _D), f32).at[:_N, :].set(x)
    src = edge_index[0]
    dst = edge_index[1]
    srcp = jnp.concatenate(
        [src, jnp.zeros((_E_PAD - _E,), jnp.int32)]).reshape(
            _NW, _CHUNKS, _CH)
    dstp = jnp.concatenate(
        [dst, jnp.full((_E_PAD - _E,), _N, jnp.int32)]).reshape(_NW, _CHUNKS, _CH)
    bi = b_in.reshape(1, _H)
    bm = b_msg.reshape(1, _H)
    bu = b_upd.reshape(1, _H)
    bo = b_out.reshape(1, _H)
    wt = W_upd[:_H]
    wb = W_upd[_H:]

    h, m = pl.pallas_call(_enc_body, out_shape=_hm_shapes())(
        xp, W_in, bi, W_msg, bm)
    out = None
    for it in range(_ITERS):
        a0, a1 = _sc_agg(m, srcp, dstp)
        if it < _ITERS - 1:
            h, m = pl.pallas_call(_upd_body, out_shape=_hm_shapes())(
                h, a0, a1, wt, wb, bu, W_msg, bm)
        else:
            out = pl.pallas_call(
                _fin_body,
                out_shape=jax.ShapeDtypeStruct((_NPAD, _H), f32))(
                    h, a0, a1, wt, wb, bu, W_out, bo)
    return out[:_N]
